# baseline (speedup 1.0000x reference)
import functools

import jax
import jax.numpy as jnp
from jax import lax
from jax.experimental import pallas as pl
from jax.experimental.pallas import tpu as pltpu
from jax.experimental.pallas import tpu_sc as plsc

N_NODES = 10000
N_EDGES = 320000
D_IN = 128
D_HID = 128
N_CLASSES = 40
D2P = 64

NC = 2
NS = 16
NW = NC * NS
NP = 10240
K = 80
NCH = N_EDGES // NW // K
NCHS = N_EDGES // NS // K
RPS = NP // NS
DDEG = 16

_mesh = lambda: plsc.VectorSubcoreMesh(core_axis_name="c", subcore_axis_name="s")


def _make_deg():
    @functools.partial(
        pl.kernel,
        mesh=_mesh(),
        compiler_params=pltpu.CompilerParams(use_tc_tiling_on_sc=False),
        out_type=jax.ShapeDtypeStruct((NC, NP, DDEG), jnp.float32),
        scratch_types=[
            pltpu.VMEM_SHARED((NP, DDEG), jnp.float32),
            pltpu.VMEM((K, DDEG), jnp.float32),
            pltpu.VMEM((NCH, K), jnp.int32),
        ],
    )
    def degk(dst_hbm, zeros_hbm, ones_hbm, out_hbm, acc, ones, dstall):
        c = lax.axis_index("c")
        s = lax.axis_index("s")
        wid = s * NC + c

        pltpu.sync_copy(ones_hbm, ones)
        pltpu.sync_copy(dst_hbm.at[wid], dstall)
        pltpu.sync_copy(zeros_hbm, acc.at[pl.ds(s * RPS, RPS)])
        plsc.subcore_barrier()

        def body(j, carry):
            pltpu.sync_copy(ones, acc.at[dstall.at[j]], add=True)
            return carry

        lax.fori_loop(0, NCH, body, 0)
        plsc.subcore_barrier()
        pltpu.sync_copy(acc.at[pl.ds(s * RPS, RPS)],
                        out_hbm.at[c, pl.ds(s * RPS, RPS)])

    return degk


NPH = 2
NCHP = NCHS // NPH


def _make_agg(d):
    delta = 2

    scratch = [pltpu.VMEM_SHARED((NP, d), jnp.float32) for _ in range(2)]
    scratch += [pltpu.VMEM((NCHP, K), jnp.int32)] * 2
    scratch += [pltpu.VMEM((K, d), jnp.float32) for _ in range(4)]
    scratch += [pltpu.SemaphoreType.DMA] * 8

    @functools.partial(
        pl.kernel,
        mesh=_mesh(),
        compiler_params=pltpu.CompilerParams(use_tc_tiling_on_sc=False),
        out_type=jax.ShapeDtypeStruct((NC, 2, NP, d), jnp.float32),
        scratch_types=scratch,
    )
    def aggk(table_hbm, src_hbm, dst_hbm, zeros_hbm, out_hbm, *scr):
        c = lax.axis_index("c")
        s = lax.axis_index("s")
        accs = scr[:2]
        srcall, dstall = scr[2], scr[3]
        rows = scr[4:8]
        semg = scr[8:12]
        sems = scr[12:16]

        offv = jnp.full((16,), c * N_NODES, jnp.int32)
        pltpu.sync_copy(zeros_hbm, accs[0].at[pl.ds(s * RPS, RPS)])
        pltpu.sync_copy(zeros_hbm, accs[1].at[pl.ds(s * RPS, RPS)])
        plsc.subcore_barrier()

        def step(j, b, wait_prev_scatter, issue_next_gather):
            b2 = (b + 2) % 4
            bw = (b + 4 - delta) % 4
            pltpu.make_async_copy(table_hbm.at[srcall.at[j]], rows[b],
                                  semg[b]).wait()
            if wait_prev_scatter:
                pltpu.make_async_copy(rows[bw],
                                      accs[bw % 2].at[dstall.at[j - delta]],
                                      sems[bw]).wait()
            pltpu.async_copy(rows[b], accs[b % 2].at[dstall.at[j]], sems[b],
                             add=True)
            if issue_next_gather:
                pltpu.async_copy(table_hbm.at[srcall.at[j + 2]], rows[b2],
                                 semg[b2])

        for p in range(NPH):
            pltpu.sync_copy(src_hbm.at[s, pl.ds(p * NCHP, NCHP)], srcall)
            pltpu.sync_copy(dst_hbm.at[s, pl.ds(p * NCHP, NCHP)], dstall)

            def addoff(i, carry):
                for t in range(K // 16):
                    sl = pl.ds(16 * t, 16)
                    srcall[i, sl] = srcall[i, sl] + offv
                return carry

            lax.fori_loop(0, NCHP, addoff, 0)

            pltpu.async_copy(table_hbm.at[srcall.at[0]], rows[0], semg[0])
            pltpu.async_copy(table_hbm.at[srcall.at[1]], rows[1], semg[1])
            for j in range(4):
                step(j, j, j >= delta, True)
            ngroup = (NCHP - 2) // 4

            def body(g, carry):
                j0 = 4 * g
                for t in range(4):
                    step(j0 + t, t, True, True)
                return carry

            lax.fori_loop(1, ngroup, body, 0)
            for j in range(4 * ngroup, NCHP):
                step(j, j % 4, True, j + 2 < NCHP)
            for j in range(NCHP - delta, NCHP):
                pltpu.make_async_copy(rows[j % 4],
                                      accs[j % 2].at[dstall.at[j]],
                                      sems[j % 4]).wait()

        plsc.subcore_barrier()
        for i in range(2):
            pltpu.sync_copy(accs[i].at[pl.ds(s * RPS, RPS)],
                            out_hbm.at[c, i, pl.ds(s * RPS, RPS)])

    return aggk


_deg_call = _make_deg()
_agg64 = _make_agg(D_HID // 2)
_agg32 = _make_agg(D2P // 2)

BN = 1000
H1 = D_HID // 2
H2 = D2P // 2


def _b_body(d_ref, x_ref, w_ref, t_ref, n_ref):
    dg = d_ref[0, :, 0:1] + d_ref[1, :, 0:1]
    nrm = lax.rsqrt(jnp.maximum(dg, 1.0))
    t1 = jnp.dot(x_ref[...], w_ref[...],
                 preferred_element_type=jnp.float32) * nrm
    t_ref[0] = t1[:, :H1]
    t_ref[1] = t1[:, H1:]
    n_ref[...] = nrm


def _tc_b(degp, x, W1):
    return pl.pallas_call(
        _b_body,
        grid=(N_NODES // BN,),
        in_specs=[
            pl.BlockSpec((NC, BN, DDEG), lambda i: (0, i, 0)),
            pl.BlockSpec((BN, D_IN), lambda i: (i, 0)),
            pl.BlockSpec((D_IN, D_HID), lambda i: (0, 0)),
        ],
        out_specs=[
            pl.BlockSpec((NC, BN, H1), lambda i: (0, i, 0)),
            pl.BlockSpec((BN, 1), lambda i: (i, 0)),
        ],
        out_shape=[
            jax.ShapeDtypeStruct((NC, N_NODES, H1), jnp.float32),
            jax.ShapeDtypeStruct((N_NODES, 1), jnp.float32),
        ],
    )(degp, x, W1)


def _d_body(p_ref, n_ref, b1_ref, w2_ref, t2_ref):
    agg = jnp.concatenate(
        [p_ref[0, 0] + p_ref[0, 1], p_ref[1, 0] + p_ref[1, 1]], axis=1)
    nrm = n_ref[...]
    h = jnp.maximum(agg * nrm + b1_ref[...], 0.0)
    t2 = jnp.dot(h, w2_ref[...], preferred_element_type=jnp.float32) * nrm
    t2_ref[0] = t2[:, :H2]
    t2_ref[1] = t2[:, H2:]


def _tc_d(p1, norm, b1r, W2p):
    return pl.pallas_call(
        _d_body,
        grid=(N_NODES // BN,),
        in_specs=[
            pl.BlockSpec((NC, 2, BN, H1), lambda i: (0, 0, i, 0)),
            pl.BlockSpec((BN, 1), lambda i: (i, 0)),
            pl.BlockSpec((1, D_HID), lambda i: (0, 0)),
            pl.BlockSpec((D_HID, D2P), lambda i: (0, 0)),
        ],
        out_specs=pl.BlockSpec((NC, BN, H2), lambda i: (0, i, 0)),
        out_shape=jax.ShapeDtypeStruct((NC, N_NODES, H2), jnp.float32),
    )(p1, norm, b1r, W2p)


def _f_body(q_ref, n_ref, b2_ref, o_ref):
    agg = jnp.concatenate(
        [q_ref[0, 0] + q_ref[0, 1], q_ref[1, 0] + q_ref[1, 1]], axis=1)
    o_ref[...] = (agg * n_ref[...])[:, :N_CLASSES] + b2_ref[...]


def _tc_f(p2, norm, b2r):
    return pl.pallas_call(
        _f_body,
        grid=(N_NODES // BN,),
        in_specs=[
            pl.BlockSpec((NC, 2, BN, H2), lambda i: (0, 0, i, 0)),
            pl.BlockSpec((BN, 1), lambda i: (i, 0)),
            pl.BlockSpec((1, N_CLASSES), lambda i: (0, 0)),
        ],
        out_specs=pl.BlockSpec((BN, N_CLASSES), lambda i: (i, 0)),
        out_shape=jax.ShapeDtypeStruct((N_NODES, N_CLASSES), jnp.float32),
    )(p2, norm, b2r)


def kernel(x, edge_index, W1, b1, W2, b2):
    src_s = edge_index[0].reshape(NS, NCHS, K)
    dst_s = edge_index[1].reshape(NS, NCHS, K)
    dst_w = edge_index[1].reshape(NW, NCH, K)

    zdeg = jnp.zeros((RPS, DDEG), jnp.float32)
    odeg = jnp.ones((K, DDEG), jnp.float32)
    z64 = jnp.zeros((RPS, H1), jnp.float32)
    z32 = jnp.zeros((RPS, H2), jnp.float32)

    degp = _deg_call(dst_w, zdeg, odeg)
    t1, norm = _tc_b(degp, x, W1)
    p1 = _agg64(t1.reshape(NC * N_NODES, H1), src_s, dst_s, z64)
    W2p = jnp.pad(W2, ((0, 0), (0, D2P - N_CLASSES)))
    t2 = _tc_d(p1, norm, b1.reshape(1, D_HID), W2p)
    p2 = _agg32(t2.reshape(NC * N_NODES, H2), src_s, dst_s, z32)
    out = _tc_f(p2, norm, b2.reshape(1, N_CLASSES))
    return out


# speedup vs baseline: 1.1050x; 1.1050x over previous
import functools

import jax
import jax.numpy as jnp
from jax import lax
from jax.experimental import pallas as pl
from jax.experimental.pallas import tpu as pltpu
from jax.experimental.pallas import tpu_sc as plsc

N_NODES = 10000
N_EDGES = 320000
D_IN = 128
D_HID = 128
N_CLASSES = 40
D2P = 64

NC = 2
NS = 16
NW = NC * NS
NP = 10240
K = 80
NCH = N_EDGES // NW // K
NCHS = N_EDGES // NS // K
RPS = NP // NS
DDEG = 16

_mesh = lambda: plsc.VectorSubcoreMesh(core_axis_name="c", subcore_axis_name="s")


def _make_deg():
    @functools.partial(
        pl.kernel,
        mesh=_mesh(),
        compiler_params=pltpu.CompilerParams(use_tc_tiling_on_sc=False),
        out_type=jax.ShapeDtypeStruct((NC, NP, DDEG), jnp.float32),
        scratch_types=[
            pltpu.VMEM_SHARED((NP, DDEG), jnp.float32),
            pltpu.VMEM((K, DDEG), jnp.float32),
            pltpu.VMEM((NCH, K), jnp.int32),
        ],
    )
    def degk(dst_hbm, zeros_hbm, ones_hbm, out_hbm, acc, ones, dstall):
        c = lax.axis_index("c")
        s = lax.axis_index("s")
        wid = s * NC + c

        pltpu.sync_copy(ones_hbm, ones)
        pltpu.sync_copy(dst_hbm.at[wid], dstall)
        pltpu.sync_copy(zeros_hbm, acc.at[pl.ds(s * RPS, RPS)])
        plsc.subcore_barrier()

        def body(j, carry):
            pltpu.sync_copy(ones, acc.at[dstall.at[j]], add=True)
            return carry

        lax.fori_loop(0, NCH, body, 0)
        plsc.subcore_barrier()
        pltpu.sync_copy(acc.at[pl.ds(s * RPS, RPS)],
                        out_hbm.at[c, pl.ds(s * RPS, RPS)])

    return degk


NPH = 2
NCHP = NCHS // NPH


def _make_agg(d):
    delta = 1

    scratch = [pltpu.VMEM_SHARED((NP, d), jnp.float32)]
    scratch += [pltpu.VMEM((NCHP, K), jnp.int32)] * 2
    scratch += [pltpu.VMEM((K, d), jnp.float32) for _ in range(4)]
    scratch += [pltpu.SemaphoreType.DMA] * 8

    @functools.partial(
        pl.kernel,
        mesh=_mesh(),
        compiler_params=pltpu.CompilerParams(use_tc_tiling_on_sc=False),
        out_type=jax.ShapeDtypeStruct((NC, 1, NP, d), jnp.float32),
        scratch_types=scratch,
    )
    def aggk(table_hbm, src_hbm, dst_hbm, zeros_hbm, out_hbm, *scr):
        c = lax.axis_index("c")
        s = lax.axis_index("s")
        accs = (scr[0], scr[0])
        srcall, dstall = scr[1], scr[2]
        rows = scr[3:7]
        semg = scr[7:11]
        sems = scr[11:15]

        offv = jnp.full((16,), c * N_NODES, jnp.int32)
        pltpu.sync_copy(zeros_hbm, accs[0].at[pl.ds(s * RPS, RPS)])
        plsc.subcore_barrier()

        def step(j, b, wait_prev_scatter, issue_next_gather):
            b2 = (b + 2) % 4
            bw = (b + 4 - delta) % 4
            pltpu.make_async_copy(table_hbm.at[srcall.at[j]], rows[b],
                                  semg[b]).wait()
            if wait_prev_scatter:
                pltpu.make_async_copy(rows[bw],
                                      accs[bw % 2].at[dstall.at[j - delta]],
                                      sems[bw]).wait()
            pltpu.async_copy(rows[b], accs[b % 2].at[dstall.at[j]], sems[b],
                             add=True)
            if issue_next_gather:
                pltpu.async_copy(table_hbm.at[srcall.at[j + 2]], rows[b2],
                                 semg[b2])

        for p in range(NPH):
            pltpu.sync_copy(src_hbm.at[s, pl.ds(p * NCHP, NCHP)], srcall)
            pltpu.sync_copy(dst_hbm.at[s, pl.ds(p * NCHP, NCHP)], dstall)

            def addoff(i, carry):
                for t in range(K // 16):
                    sl = pl.ds(16 * t, 16)
                    srcall[i, sl] = srcall[i, sl] + offv
                return carry

            lax.fori_loop(0, NCHP, addoff, 0)

            pltpu.async_copy(table_hbm.at[srcall.at[0]], rows[0], semg[0])
            pltpu.async_copy(table_hbm.at[srcall.at[1]], rows[1], semg[1])
            for j in range(4):
                step(j, j, j >= delta, True)
            ngroup = (NCHP - 2) // 4

            def body(g, carry):
                j0 = 4 * g
                for t in range(4):
                    step(j0 + t, t, True, True)
                return carry

            lax.fori_loop(1, ngroup, body, 0)
            for j in range(4 * ngroup, NCHP):
                step(j, j % 4, True, j + 2 < NCHP)
            for j in range(NCHP - delta, NCHP):
                pltpu.make_async_copy(rows[j % 4],
                                      accs[j % 2].at[dstall.at[j]],
                                      sems[j % 4]).wait()

        plsc.subcore_barrier()
        pltpu.sync_copy(accs[0].at[pl.ds(s * RPS, RPS)],
                        out_hbm.at[c, 0, pl.ds(s * RPS, RPS)])

    return aggk


_deg_call = _make_deg()
_agg64 = _make_agg(D_HID // 2)
_agg32 = _make_agg(D2P // 2)

BN = 1000
H1 = D_HID // 2
H2 = D2P // 2


def _b_body(d_ref, x_ref, w_ref, t_ref, n_ref):
    dg = d_ref[0, :, 0:1] + d_ref[1, :, 0:1]
    nrm = lax.rsqrt(jnp.maximum(dg, 1.0))
    t1 = jnp.dot(x_ref[...], w_ref[...],
                 preferred_element_type=jnp.float32) * nrm
    t_ref[0] = t1[:, :H1]
    t_ref[1] = t1[:, H1:]
    n_ref[...] = nrm


def _tc_b(degp, x, W1):
    return pl.pallas_call(
        _b_body,
        grid=(N_NODES // BN,),
        in_specs=[
            pl.BlockSpec((NC, BN, DDEG), lambda i: (0, i, 0)),
            pl.BlockSpec((BN, D_IN), lambda i: (i, 0)),
            pl.BlockSpec((D_IN, D_HID), lambda i: (0, 0)),
        ],
        out_specs=[
            pl.BlockSpec((NC, BN, H1), lambda i: (0, i, 0)),
            pl.BlockSpec((BN, 1), lambda i: (i, 0)),
        ],
        out_shape=[
            jax.ShapeDtypeStruct((NC, N_NODES, H1), jnp.float32),
            jax.ShapeDtypeStruct((N_NODES, 1), jnp.float32),
        ],
    )(degp, x, W1)


def _d_body(p_ref, n_ref, b1_ref, w2_ref, t2_ref):
    agg = jnp.concatenate([p_ref[0, 0], p_ref[1, 0]], axis=1)
    nrm = n_ref[...]
    h = jnp.maximum(agg * nrm + b1_ref[...], 0.0)
    t2 = jnp.dot(h, w2_ref[...], preferred_element_type=jnp.float32) * nrm
    t2_ref[0] = t2[:, :H2]
    t2_ref[1] = t2[:, H2:]


def _tc_d(p1, norm, b1r, W2p):
    return pl.pallas_call(
        _d_body,
        grid=(N_NODES // BN,),
        in_specs=[
            pl.BlockSpec((NC, 1, BN, H1), lambda i: (0, 0, i, 0)),
            pl.BlockSpec((BN, 1), lambda i: (i, 0)),
            pl.BlockSpec((1, D_HID), lambda i: (0, 0)),
            pl.BlockSpec((D_HID, D2P), lambda i: (0, 0)),
        ],
        out_specs=pl.BlockSpec((NC, BN, H2), lambda i: (0, i, 0)),
        out_shape=jax.ShapeDtypeStruct((NC, N_NODES, H2), jnp.float32),
    )(p1, norm, b1r, W2p)


def _f_body(q_ref, n_ref, b2_ref, o_ref):
    agg = jnp.concatenate([q_ref[0, 0], q_ref[1, 0]], axis=1)
    o_ref[...] = (agg * n_ref[...])[:, :N_CLASSES] + b2_ref[...]


def _tc_f(p2, norm, b2r):
    return pl.pallas_call(
        _f_body,
        grid=(N_NODES // BN,),
        in_specs=[
            pl.BlockSpec((NC, 1, BN, H2), lambda i: (0, 0, i, 0)),
            pl.BlockSpec((BN, 1), lambda i: (i, 0)),
            pl.BlockSpec((1, N_CLASSES), lambda i: (0, 0)),
        ],
        out_specs=pl.BlockSpec((BN, N_CLASSES), lambda i: (i, 0)),
        out_shape=jax.ShapeDtypeStruct((N_NODES, N_CLASSES), jnp.float32),
    )(p2, norm, b2r)


def kernel(x, edge_index, W1, b1, W2, b2):
    src_s = edge_index[0].reshape(NS, NCHS, K)
    dst_s = edge_index[1].reshape(NS, NCHS, K)
    dst_w = edge_index[1].reshape(NW, NCH, K)

    zdeg = jnp.zeros((RPS, DDEG), jnp.float32)
    odeg = jnp.ones((K, DDEG), jnp.float32)
    z64 = jnp.zeros((RPS, H1), jnp.float32)
    z32 = jnp.zeros((RPS, H2), jnp.float32)

    degp = _deg_call(dst_w, zdeg, odeg)
    t1, norm = _tc_b(degp, x, W1)
    p1 = _agg64(t1.reshape(NC * N_NODES, H1), src_s, dst_s, z64)
    W2p = jnp.pad(W2, ((0, 0), (0, D2P - N_CLASSES)))
    t2 = _tc_d(p1, norm, b1.reshape(1, D_HID), W2p)
    p2 = _agg32(t2.reshape(NC * N_NODES, H2), src_s, dst_s, z32)
    out = _tc_f(p2, norm, b2.reshape(1, N_CLASSES))
    return out


# speedup vs baseline: 1.1178x; 1.0115x over previous
import functools

import jax
import jax.numpy as jnp
from jax import lax
from jax.experimental import pallas as pl
from jax.experimental.pallas import tpu as pltpu
from jax.experimental.pallas import tpu_sc as plsc

N_NODES = 10000
N_EDGES = 320000
D_IN = 128
D_HID = 128
N_CLASSES = 40
D2P = 64

NC = 2
NS = 16
NW = NC * NS
NP = 10240
K = 80
NCH = N_EDGES // NW // K
NCHS = N_EDGES // NS // K
RPS = NP // NS
DDEG = 16

_mesh = lambda: plsc.VectorSubcoreMesh(core_axis_name="c", subcore_axis_name="s")


def _make_deg():
    @functools.partial(
        pl.kernel,
        mesh=_mesh(),
        compiler_params=pltpu.CompilerParams(use_tc_tiling_on_sc=False),
        out_type=jax.ShapeDtypeStruct((NC, NP, DDEG), jnp.float32),
        scratch_types=[
            pltpu.VMEM_SHARED((NP, DDEG), jnp.float32),
            pltpu.VMEM((K, DDEG), jnp.float32),
            pltpu.VMEM((NCH, K), jnp.int32),
        ],
    )
    def degk(dst_hbm, zeros_hbm, ones_hbm, out_hbm, acc, ones, dstall):
        c = lax.axis_index("c")
        s = lax.axis_index("s")
        wid = s * NC + c

        pltpu.sync_copy(ones_hbm, ones)
        pltpu.sync_copy(dst_hbm.at[wid], dstall)
        pltpu.sync_copy(zeros_hbm, acc.at[pl.ds(s * RPS, RPS)])
        plsc.subcore_barrier()

        def body(j, carry):
            pltpu.sync_copy(ones, acc.at[dstall.at[j]], add=True)
            return carry

        lax.fori_loop(0, NCH, body, 0)
        plsc.subcore_barrier()
        pltpu.sync_copy(acc.at[pl.ds(s * RPS, RPS)],
                        out_hbm.at[c, pl.ds(s * RPS, RPS)])

    return degk


NPH = 1
NCHP = NCHS // NPH


def _make_agg(d):
    delta = 1

    scratch = [pltpu.VMEM_SHARED((NP, d), jnp.float32)]
    scratch += [pltpu.VMEM((NCHP, K), jnp.int32)] * 2
    scratch += [pltpu.VMEM((K, d), jnp.float32) for _ in range(4)]
    scratch += [pltpu.SemaphoreType.DMA] * 8

    @functools.partial(
        pl.kernel,
        mesh=_mesh(),
        compiler_params=pltpu.CompilerParams(use_tc_tiling_on_sc=False),
        out_type=jax.ShapeDtypeStruct((NC, 1, NP, d), jnp.float32),
        scratch_types=scratch,
    )
    def aggk(table_hbm, src_hbm, dst_hbm, zeros_hbm, out_hbm, *scr):
        c = lax.axis_index("c")
        s = lax.axis_index("s")
        accs = (scr[0], scr[0])
        srcall, dstall = scr[1], scr[2]
        rows = scr[3:7]
        semg = scr[7:11]
        sems = scr[11:15]

        offv = jnp.full((16,), c * N_NODES, jnp.int32)
        pltpu.sync_copy(zeros_hbm, accs[0].at[pl.ds(s * RPS, RPS)])
        plsc.subcore_barrier()

        def step(j, b, wait_prev_scatter, issue_next_gather):
            b2 = (b + 2) % 4
            bw = (b + 4 - delta) % 4
            pltpu.make_async_copy(table_hbm.at[srcall.at[j]], rows[b],
                                  semg[b]).wait()
            if wait_prev_scatter:
                pltpu.make_async_copy(rows[bw],
                                      accs[bw % 2].at[dstall.at[j - delta]],
                                      sems[bw]).wait()
            pltpu.async_copy(rows[b], accs[b % 2].at[dstall.at[j]], sems[b],
                             add=True)
            if issue_next_gather:
                pltpu.async_copy(table_hbm.at[srcall.at[j + 2]], rows[b2],
                                 semg[b2])

        for p in range(NPH):
            pltpu.sync_copy(src_hbm.at[s, pl.ds(p * NCHP, NCHP)], srcall)
            pltpu.sync_copy(dst_hbm.at[s, pl.ds(p * NCHP, NCHP)], dstall)

            def addoff(i, carry):
                for t in range(K // 16):
                    sl = pl.ds(16 * t, 16)
                    srcall[i, sl] = srcall[i, sl] + offv
                return carry

            lax.fori_loop(0, NCHP, addoff, 0)

            pltpu.async_copy(table_hbm.at[srcall.at[0]], rows[0], semg[0])
            pltpu.async_copy(table_hbm.at[srcall.at[1]], rows[1], semg[1])
            for j in range(4):
                step(j, j, j >= delta, True)
            ngroup = (NCHP - 2) // 4

            def body(g, carry):
                j0 = 4 * g
                for t in range(4):
                    step(j0 + t, t, True, True)
                return carry

            lax.fori_loop(1, ngroup, body, 0)
            for j in range(4 * ngroup, NCHP):
                step(j, j % 4, True, j + 2 < NCHP)
            for j in range(NCHP - delta, NCHP):
                pltpu.make_async_copy(rows[j % 4],
                                      accs[j % 2].at[dstall.at[j]],
                                      sems[j % 4]).wait()

        plsc.subcore_barrier()
        pltpu.sync_copy(accs[0].at[pl.ds(s * RPS, RPS)],
                        out_hbm.at[c, 0, pl.ds(s * RPS, RPS)])

    return aggk


_deg_call = _make_deg()
_agg64 = _make_agg(D_HID // 2)
_agg32 = _make_agg(D2P // 2)

BN = 1000
H1 = D_HID // 2
H2 = D2P // 2


def _b_body(d_ref, x_ref, w_ref, t_ref, n_ref):
    dg = d_ref[0, :, 0:1] + d_ref[1, :, 0:1]
    nrm = lax.rsqrt(jnp.maximum(dg, 1.0))
    t1 = jnp.dot(x_ref[...], w_ref[...],
                 preferred_element_type=jnp.float32) * nrm
    t_ref[0] = t1[:, :H1]
    t_ref[1] = t1[:, H1:]
    n_ref[...] = nrm


def _tc_b(degp, x, W1):
    return pl.pallas_call(
        _b_body,
        grid=(N_NODES // BN,),
        in_specs=[
            pl.BlockSpec((NC, BN, DDEG), lambda i: (0, i, 0)),
            pl.BlockSpec((BN, D_IN), lambda i: (i, 0)),
            pl.BlockSpec((D_IN, D_HID), lambda i: (0, 0)),
        ],
        out_specs=[
            pl.BlockSpec((NC, BN, H1), lambda i: (0, i, 0)),
            pl.BlockSpec((BN, 1), lambda i: (i, 0)),
        ],
        out_shape=[
            jax.ShapeDtypeStruct((NC, N_NODES, H1), jnp.float32),
            jax.ShapeDtypeStruct((N_NODES, 1), jnp.float32),
        ],
    )(degp, x, W1)


def _d_body(p_ref, n_ref, b1_ref, w2_ref, t2_ref):
    agg = jnp.concatenate([p_ref[0, 0], p_ref[1, 0]], axis=1)
    nrm = n_ref[...]
    h = jnp.maximum(agg * nrm + b1_ref[...], 0.0)
    t2 = jnp.dot(h, w2_ref[...], preferred_element_type=jnp.float32) * nrm
    t2_ref[0] = t2[:, :H2]
    t2_ref[1] = t2[:, H2:]


def _tc_d(p1, norm, b1r, W2p):
    return pl.pallas_call(
        _d_body,
        grid=(N_NODES // BN,),
        in_specs=[
            pl.BlockSpec((NC, 1, BN, H1), lambda i: (0, 0, i, 0)),
            pl.BlockSpec((BN, 1), lambda i: (i, 0)),
            pl.BlockSpec((1, D_HID), lambda i: (0, 0)),
            pl.BlockSpec((D_HID, D2P), lambda i: (0, 0)),
        ],
        out_specs=pl.BlockSpec((NC, BN, H2), lambda i: (0, i, 0)),
        out_shape=jax.ShapeDtypeStruct((NC, N_NODES, H2), jnp.float32),
    )(p1, norm, b1r, W2p)


def _f_body(q_ref, n_ref, b2_ref, o_ref):
    agg = jnp.concatenate([q_ref[0, 0], q_ref[1, 0]], axis=1)
    o_ref[...] = (agg * n_ref[...])[:, :N_CLASSES] + b2_ref[...]


def _tc_f(p2, norm, b2r):
    return pl.pallas_call(
        _f_body,
        grid=(N_NODES // BN,),
        in_specs=[
            pl.BlockSpec((NC, 1, BN, H2), lambda i: (0, 0, i, 0)),
            pl.BlockSpec((BN, 1), lambda i: (i, 0)),
            pl.BlockSpec((1, N_CLASSES), lambda i: (0, 0)),
        ],
        out_specs=pl.BlockSpec((BN, N_CLASSES), lambda i: (i, 0)),
        out_shape=jax.ShapeDtypeStruct((N_NODES, N_CLASSES), jnp.float32),
    )(p2, norm, b2r)


def kernel(x, edge_index, W1, b1, W2, b2):
    src_s = edge_index[0].reshape(NS, NCHS, K)
    dst_s = edge_index[1].reshape(NS, NCHS, K)
    dst_w = edge_index[1].reshape(NW, NCH, K)

    zdeg = jnp.zeros((RPS, DDEG), jnp.float32)
    odeg = jnp.ones((K, DDEG), jnp.float32)
    z64 = jnp.zeros((RPS, H1), jnp.float32)
    z32 = jnp.zeros((RPS, H2), jnp.float32)

    degp = _deg_call(dst_w, zdeg, odeg)
    t1, norm = _tc_b(degp, x, W1)
    p1 = _agg64(t1.reshape(NC * N_NODES, H1), src_s, dst_s, z64)
    W2p = jnp.pad(W2, ((0, 0), (0, D2P - N_CLASSES)))
    t2 = _tc_d(p1, norm, b1.reshape(1, D_HID), W2p)
    p2 = _agg32(t2.reshape(NC * N_NODES, H2), src_s, dst_s, z32)
    out = _tc_f(p2, norm, b2.reshape(1, N_CLASSES))
    return out


# speedup vs baseline: 1.1466x; 1.0258x over previous
import functools

import jax
import jax.numpy as jnp
from jax import lax
from jax.experimental import pallas as pl
from jax.experimental.pallas import tpu as pltpu
from jax.experimental.pallas import tpu_sc as plsc

N_NODES = 10000
N_EDGES = 320000
D_IN = 128
D_HID = 128
N_CLASSES = 40
D2P = 64

NC = 2
NS = 16
NW = NC * NS
NP = 10240
K = 80
NCH = N_EDGES // NW // K
NCHS = N_EDGES // NS // K
RPS = NP // NS
DDEG = 16

_mesh = lambda: plsc.VectorSubcoreMesh(core_axis_name="c", subcore_axis_name="s")

ZR = 128
NCOPY = RPS // ZR


def _zero_fill(zbuf, rows, d):
    zv = jnp.zeros((16,), jnp.float32)

    def body(i, carry):
        for j in range(d // 16):
            zbuf[i, pl.ds(j * 16, 16)] = zv
        return carry

    lax.fori_loop(0, rows, body, 0)


def _make_deg():
    @functools.partial(
        pl.kernel,
        mesh=_mesh(),
        compiler_params=pltpu.CompilerParams(use_tc_tiling_on_sc=False),
        out_type=jax.ShapeDtypeStruct((NC, NP, DDEG), jnp.float32),
        scratch_types=[
            pltpu.VMEM_SHARED((NP, DDEG), jnp.float32),
            pltpu.VMEM((ZR, DDEG), jnp.float32),
            pltpu.VMEM((K, DDEG), jnp.float32),
            pltpu.VMEM((NCH, K), jnp.int32),
        ],
    )
    def degk(dst_hbm, out_hbm, acc, zbuf, ones, dstall):
        c = lax.axis_index("c")
        s = lax.axis_index("s")
        wid = s * NC + c

        _zero_fill(zbuf, ZR, DDEG)
        ov = jnp.ones((16,), jnp.float32)

        def fill_ones(i, carry):
            ones[i, pl.ds(0, DDEG)] = ov
            return carry

        lax.fori_loop(0, K, fill_ones, 0)
        pltpu.sync_copy(dst_hbm.at[wid], dstall)
        for r in range(NCOPY):
            pltpu.sync_copy(zbuf, acc.at[pl.ds(s * RPS + r * ZR, ZR)])
        plsc.subcore_barrier()

        def body(j, carry):
            pltpu.sync_copy(ones, acc.at[dstall.at[j]], add=True)
            return carry

        lax.fori_loop(0, NCH, body, 0)
        plsc.subcore_barrier()
        pltpu.sync_copy(acc.at[pl.ds(s * RPS, RPS)],
                        out_hbm.at[c, pl.ds(s * RPS, RPS)])

    return degk


NPH = 1
NCHP = NCHS // NPH


def _make_agg(d):
    delta = 1

    scratch = [pltpu.VMEM_SHARED((NP, d), jnp.float32)]
    scratch += [pltpu.VMEM((ZR, d), jnp.float32)]
    scratch += [pltpu.VMEM((NCHP, K), jnp.int32)] * 2
    scratch += [pltpu.VMEM((K, d), jnp.float32) for _ in range(4)]
    scratch += [pltpu.SemaphoreType.DMA] * 8

    @functools.partial(
        pl.kernel,
        mesh=_mesh(),
        compiler_params=pltpu.CompilerParams(use_tc_tiling_on_sc=False),
        out_type=jax.ShapeDtypeStruct((NC, 1, NP, d), jnp.float32),
        scratch_types=scratch,
    )
    def aggk(table_hbm, src_hbm, dst_hbm, out_hbm, *scr):
        c = lax.axis_index("c")
        s = lax.axis_index("s")
        accs = (scr[0], scr[0])
        zbuf = scr[1]
        srcall, dstall = scr[2], scr[3]
        rows = scr[4:8]
        semg = scr[8:12]
        sems = scr[12:16]

        offv = jnp.full((16,), c * N_NODES, jnp.int32)
        _zero_fill(zbuf, ZR, d)
        for r in range(NCOPY):
            pltpu.sync_copy(zbuf, accs[0].at[pl.ds(s * RPS + r * ZR, ZR)])
        plsc.subcore_barrier()

        def step(j, b, wait_prev_scatter, issue_next_gather):
            b2 = (b + 2) % 4
            bw = (b + 4 - delta) % 4
            pltpu.make_async_copy(table_hbm.at[srcall.at[j]], rows[b],
                                  semg[b]).wait()
            if wait_prev_scatter:
                pltpu.make_async_copy(rows[bw],
                                      accs[bw % 2].at[dstall.at[j - delta]],
                                      sems[bw]).wait()
            pltpu.async_copy(rows[b], accs[b % 2].at[dstall.at[j]], sems[b],
                             add=True)
            if issue_next_gather:
                pltpu.async_copy(table_hbm.at[srcall.at[j + 2]], rows[b2],
                                 semg[b2])

        for p in range(NPH):
            pltpu.sync_copy(src_hbm.at[s, pl.ds(p * NCHP, NCHP)], srcall)
            pltpu.sync_copy(dst_hbm.at[s, pl.ds(p * NCHP, NCHP)], dstall)

            def addoff(i, carry):
                for t in range(K // 16):
                    sl = pl.ds(16 * t, 16)
                    srcall[i, sl] = srcall[i, sl] + offv
                return carry

            lax.fori_loop(0, NCHP, addoff, 0)

            pltpu.async_copy(table_hbm.at[srcall.at[0]], rows[0], semg[0])
            pltpu.async_copy(table_hbm.at[srcall.at[1]], rows[1], semg[1])
            for j in range(4):
                step(j, j, j >= delta, True)
            ngroup = (NCHP - 2) // 4

            def body(g, carry):
                j0 = 4 * g
                for t in range(4):
                    step(j0 + t, t, True, True)
                return carry

            lax.fori_loop(1, ngroup, body, 0)
            for j in range(4 * ngroup, NCHP):
                step(j, j % 4, True, j + 2 < NCHP)
            for j in range(NCHP - delta, NCHP):
                pltpu.make_async_copy(rows[j % 4],
                                      accs[j % 2].at[dstall.at[j]],
                                      sems[j % 4]).wait()

        plsc.subcore_barrier()
        pltpu.sync_copy(accs[0].at[pl.ds(s * RPS, RPS)],
                        out_hbm.at[c, 0, pl.ds(s * RPS, RPS)])

    return aggk


_deg_call = _make_deg()
_agg64 = _make_agg(D_HID // 2)
_agg32 = _make_agg(D2P // 2)

BN = 1000
H1 = D_HID // 2
H2 = D2P // 2


def _b_body(d_ref, x_ref, w_ref, t_ref, n_ref):
    dg = d_ref[0, :, 0:1] + d_ref[1, :, 0:1]
    nrm = lax.rsqrt(jnp.maximum(dg, 1.0))
    t1 = jnp.dot(x_ref[...], w_ref[...],
                 preferred_element_type=jnp.float32) * nrm
    t_ref[0] = t1[:, :H1]
    t_ref[1] = t1[:, H1:]
    n_ref[...] = nrm


def _tc_b(degp, x, W1):
    return pl.pallas_call(
        _b_body,
        grid=(N_NODES // BN,),
        in_specs=[
            pl.BlockSpec((NC, BN, DDEG), lambda i: (0, i, 0)),
            pl.BlockSpec((BN, D_IN), lambda i: (i, 0)),
            pl.BlockSpec((D_IN, D_HID), lambda i: (0, 0)),
        ],
        out_specs=[
            pl.BlockSpec((NC, BN, H1), lambda i: (0, i, 0)),
            pl.BlockSpec((BN, 1), lambda i: (i, 0)),
        ],
        out_shape=[
            jax.ShapeDtypeStruct((NC, N_NODES, H1), jnp.float32),
            jax.ShapeDtypeStruct((N_NODES, 1), jnp.float32),
        ],
    )(degp, x, W1)


def _d_body(p_ref, n_ref, b1_ref, w2_ref, t2_ref):
    agg = jnp.concatenate([p_ref[0, 0], p_ref[1, 0]], axis=1)
    nrm = n_ref[...]
    h = jnp.maximum(agg * nrm + b1_ref[...], 0.0)
    t2 = jnp.dot(h, w2_ref[...], preferred_element_type=jnp.float32) * nrm
    t2_ref[0] = t2[:, :H2]
    t2_ref[1] = t2[:, H2:]


def _tc_d(p1, norm, b1r, W2p):
    return pl.pallas_call(
        _d_body,
        grid=(N_NODES // BN,),
        in_specs=[
            pl.BlockSpec((NC, 1, BN, H1), lambda i: (0, 0, i, 0)),
            pl.BlockSpec((BN, 1), lambda i: (i, 0)),
            pl.BlockSpec((1, D_HID), lambda i: (0, 0)),
            pl.BlockSpec((D_HID, D2P), lambda i: (0, 0)),
        ],
        out_specs=pl.BlockSpec((NC, BN, H2), lambda i: (0, i, 0)),
        out_shape=jax.ShapeDtypeStruct((NC, N_NODES, H2), jnp.float32),
    )(p1, norm, b1r, W2p)


def _f_body(q_ref, n_ref, b2_ref, o_ref):
    agg = jnp.concatenate([q_ref[0, 0], q_ref[1, 0]], axis=1)
    o_ref[...] = (agg * n_ref[...])[:, :N_CLASSES] + b2_ref[...]


def _tc_f(p2, norm, b2r):
    return pl.pallas_call(
        _f_body,
        grid=(N_NODES // BN,),
        in_specs=[
            pl.BlockSpec((NC, 1, BN, H2), lambda i: (0, 0, i, 0)),
            pl.BlockSpec((BN, 1), lambda i: (i, 0)),
            pl.BlockSpec((1, N_CLASSES), lambda i: (0, 0)),
        ],
        out_specs=pl.BlockSpec((BN, N_CLASSES), lambda i: (i, 0)),
        out_shape=jax.ShapeDtypeStruct((N_NODES, N_CLASSES), jnp.float32),
    )(p2, norm, b2r)


def kernel(x, edge_index, W1, b1, W2, b2):
    src_s = edge_index[0].reshape(NS, NCHS, K)
    dst_s = edge_index[1].reshape(NS, NCHS, K)
    dst_w = edge_index[1].reshape(NW, NCH, K)

    degp = _deg_call(dst_w)
    t1, norm = _tc_b(degp, x, W1)
    p1 = _agg64(t1.reshape(NC * N_NODES, H1), src_s, dst_s)
    W2p = jnp.pad(W2, ((0, 0), (0, D2P - N_CLASSES)))
    t2 = _tc_d(p1, norm, b1.reshape(1, D_HID), W2p)
    p2 = _agg32(t2.reshape(NC * N_NODES, H2), src_s, dst_s)
    out = _tc_f(p2, norm, b2.reshape(1, N_CLASSES))
    return out
